# baseline (device time: 150209 ns/iter reference)
import jax
import jax.numpy as jnp
from jax import lax
from jax.experimental import pallas as pl
from jax.experimental.pallas import tpu as pltpu

N_DEV = 4
N_LOC_E = 2
CAP = 384
NC = N_DEV * CAP


def kernel(x, assign, W1, W2):
    t, d = x.shape
    _, _, f = W1.shape
    xb = x.astype(jnp.bfloat16)
    w1b = W1.astype(jnp.bfloat16)
    w2b = W2.astype(jnp.bfloat16)

    dest = assign // N_LOC_E
    onehot = jax.nn.one_hot(dest, N_DEV, dtype=jnp.int32)
    pos = jnp.cumsum(onehot, axis=0)[jnp.arange(t), dest] - 1
    k = dest * CAP + pos
    G = jnp.zeros((NC, t), jnp.bfloat16).at[k, jnp.arange(t)].set(1)
    Gt = jnp.zeros((t, NC), jnp.bfloat16).at[jnp.arange(t), k].set(1)
    asend = jnp.full((NC, 1), -1, jnp.int32).at[k, 0].set(assign)
    asend = asend.reshape(N_DEV, CAP, 1)

    def body(x_ref, g_ref, gt_ref, as_ref, w1_ref, w2_ref, out_ref,
             xsend, xr, ar, pbuf, rrec,
             sx, rx, sa, ra, sp, rp):
        my = lax.axis_index("i")

        barrier = pltpu.get_barrier_semaphore()
        for off in range(1, N_DEV):
            pl.semaphore_signal(
                barrier, inc=1,
                device_id=((my + off) % N_DEV,),
                device_id_type=pl.DeviceIdType.MESH,
            )
        pl.semaphore_wait(barrier, N_DEV - 1)

        xsend[:] = jnp.dot(
            g_ref[:], x_ref[:], preferred_element_type=jnp.float32
        ).astype(jnp.bfloat16)
        xr[pl.ds(my * CAP, CAP), :] = xsend[pl.ds(my * CAP, CAP), :]
        ar[my] = as_ref[my]

        sends = []
        for off in range(1, N_DEV):
            dst = (my + off) % N_DEV
            cx = pltpu.make_async_remote_copy(
                src_ref=xsend.at[pl.ds(dst * CAP, CAP)],
                dst_ref=xr.at[pl.ds(my * CAP, CAP)],
                send_sem=sx.at[off], recv_sem=rx.at[my],
                device_id=(dst,), device_id_type=pl.DeviceIdType.MESH,
            )
            cx.start()
            ca = pltpu.make_async_remote_copy(
                src_ref=as_ref.at[dst], dst_ref=ar.at[my],
                send_sem=sa.at[off], recv_sem=ra.at[my],
                device_id=(dst,), device_id_type=pl.DeviceIdType.MESH,
            )
            ca.start()
            sends += [cx, ca]

        def recv_wait_rows(buf, sems, s):
            rcv = pltpu.make_async_remote_copy(
                src_ref=buf.at[pl.ds(s * CAP, CAP)],
                dst_ref=buf.at[pl.ds(s * CAP, CAP)],
                send_sem=sems.at[s], recv_sem=sems.at[s],
                device_id=(my,), device_id_type=pl.DeviceIdType.MESH,
            )
            rcv.wait_recv()

        def recv_wait(buf, sems, s):
            rcv = pltpu.make_async_remote_copy(
                src_ref=buf.at[s], dst_ref=buf.at[s],
                send_sem=sems.at[s], recv_sem=sems.at[s],
                device_id=(my,), device_id_type=pl.DeviceIdType.MESH,
            )
            rcv.wait_recv()

        def bucket_partial(src):
            xs = xr[pl.ds(src * CAP, CAP), :]
            asg = ar[src]
            acc = None
            for le in range(N_LOC_E):
                e = my * N_LOC_E + le
                xm = jnp.where(asg == e, xs, jnp.bfloat16(0.0))
                h = jnp.dot(xm, w1_ref[le], preferred_element_type=jnp.float32)
                h = jnp.maximum(h, 0.0).astype(jnp.bfloat16)
                p = jnp.dot(h, w2_ref[le], preferred_element_type=jnp.float32)
                acc = p if acc is None else acc + p
            return acc.astype(jnp.bfloat16)

        rrec[pl.ds(my * CAP, CAP), :] = bucket_partial(my)

        for off in range(1, N_DEV):
            src = (my + off) % N_DEV
            recv_wait_rows(xr, rx, src)
            recv_wait(ar, ra, src)
            pbuf[off - 1] = bucket_partial(src)
            cp = pltpu.make_async_remote_copy(
                src_ref=pbuf.at[off - 1],
                dst_ref=rrec.at[pl.ds(my * CAP, CAP)],
                send_sem=sp.at[off], recv_sem=rp.at[my],
                device_id=(src,), device_id_type=pl.DeviceIdType.MESH,
            )
            cp.start()
            sends.append(cp)

        for off in range(1, N_DEV):
            s = (my + off) % N_DEV
            recv_wait_rows(rrec, rp, s)
        out_ref[:] = jnp.dot(
            gt_ref[:], rrec[:], preferred_element_type=jnp.float32
        )

        for c in sends:
            c.wait_send()

    return pl.pallas_call(
        body,
        out_shape=jax.ShapeDtypeStruct((t, d), jnp.float32),
        in_specs=[pl.BlockSpec(memory_space=pltpu.VMEM)] * 6,
        out_specs=pl.BlockSpec(memory_space=pltpu.VMEM),
        scratch_shapes=[
            pltpu.VMEM((NC, d), jnp.bfloat16),
            pltpu.VMEM((NC, d), jnp.bfloat16),
            pltpu.VMEM((N_DEV, CAP, 1), jnp.int32),
            pltpu.VMEM((N_DEV - 1, CAP, d), jnp.bfloat16),
            pltpu.VMEM((NC, d), jnp.bfloat16),
            pltpu.SemaphoreType.DMA((N_DEV,)),
            pltpu.SemaphoreType.DMA((N_DEV,)),
            pltpu.SemaphoreType.DMA((N_DEV,)),
            pltpu.SemaphoreType.DMA((N_DEV,)),
            pltpu.SemaphoreType.DMA((N_DEV,)),
            pltpu.SemaphoreType.DMA((N_DEV,)),
        ],
        compiler_params=pltpu.CompilerParams(
            collective_id=0,
            vmem_limit_bytes=48 * 1024 * 1024,
        ),
    )(xb, G, Gt, asend, w1b, w2b)


# device time: 108051 ns/iter; 1.3902x vs baseline; 1.3902x over previous
import jax
import jax.numpy as jnp
from jax import lax
from jax.experimental import pallas as pl
from jax.experimental.pallas import tpu as pltpu

N_DEV = 4
N_LOC_E = 2
CAP = 384
NC = N_DEV * CAP


def kernel(x, assign, W1, W2):
    t, d = x.shape
    _, _, f = W1.shape
    xb = x.astype(jnp.bfloat16)
    w1b = W1.astype(jnp.bfloat16)
    w2b = W2.astype(jnp.bfloat16)

    dest = assign // N_LOC_E
    onehot = (dest[:, None] == jnp.arange(N_DEV)[None, :]).astype(jnp.int32)
    pos = jnp.sum(onehot * jnp.cumsum(onehot, axis=0), axis=1) - 1
    k = dest * CAP + pos
    iota = jnp.arange(NC)
    G = (k[None, :] == iota[:, None]).astype(jnp.bfloat16)
    Gt = (k[:, None] == iota[None, :]).astype(jnp.bfloat16)
    asend = (
        jnp.dot(G.astype(jnp.float32), (assign + 1).astype(jnp.float32))
        .astype(jnp.int32) - 1
    ).reshape(N_DEV, CAP, 1)

    def body(x_ref, g_ref, gt_ref, as_ref, w1_ref, w2_ref, out_ref,
             xsend, xr, ar, pbuf, rrec,
             sx, rx, sa, ra, sp, rp):
        my = lax.axis_index("i")

        barrier = pltpu.get_barrier_semaphore()
        for off in range(1, N_DEV):
            pl.semaphore_signal(
                barrier, inc=1,
                device_id=((my + off) % N_DEV,),
                device_id_type=pl.DeviceIdType.MESH,
            )
        pl.semaphore_wait(barrier, N_DEV - 1)

        xsend[:] = jnp.dot(
            g_ref[:], x_ref[:], preferred_element_type=jnp.float32
        ).astype(jnp.bfloat16)
        xr[pl.ds(my * CAP, CAP), :] = xsend[pl.ds(my * CAP, CAP), :]
        ar[my] = as_ref[my]

        sends = []
        for off in range(1, N_DEV):
            dst = (my + off) % N_DEV
            cx = pltpu.make_async_remote_copy(
                src_ref=xsend.at[pl.ds(dst * CAP, CAP)],
                dst_ref=xr.at[pl.ds(my * CAP, CAP)],
                send_sem=sx.at[off], recv_sem=rx.at[my],
                device_id=(dst,), device_id_type=pl.DeviceIdType.MESH,
            )
            cx.start()
            ca = pltpu.make_async_remote_copy(
                src_ref=as_ref.at[dst], dst_ref=ar.at[my],
                send_sem=sa.at[off], recv_sem=ra.at[my],
                device_id=(dst,), device_id_type=pl.DeviceIdType.MESH,
            )
            ca.start()
            sends += [cx, ca]

        def recv_wait_rows(buf, sems, s):
            rcv = pltpu.make_async_remote_copy(
                src_ref=buf.at[pl.ds(s * CAP, CAP)],
                dst_ref=buf.at[pl.ds(s * CAP, CAP)],
                send_sem=sems.at[s], recv_sem=sems.at[s],
                device_id=(my,), device_id_type=pl.DeviceIdType.MESH,
            )
            rcv.wait_recv()

        def recv_wait(buf, sems, s):
            rcv = pltpu.make_async_remote_copy(
                src_ref=buf.at[s], dst_ref=buf.at[s],
                send_sem=sems.at[s], recv_sem=sems.at[s],
                device_id=(my,), device_id_type=pl.DeviceIdType.MESH,
            )
            rcv.wait_recv()

        def bucket_partial(src):
            xs = xr[pl.ds(src * CAP, CAP), :]
            asg = ar[src]
            acc = None
            for le in range(N_LOC_E):
                e = my * N_LOC_E + le
                xm = jnp.where(asg == e, xs, jnp.bfloat16(0.0))
                h = jnp.dot(xm, w1_ref[le], preferred_element_type=jnp.float32)
                h = jnp.maximum(h, 0.0).astype(jnp.bfloat16)
                p = jnp.dot(h, w2_ref[le], preferred_element_type=jnp.float32)
                acc = p if acc is None else acc + p
            return acc.astype(jnp.bfloat16)

        rrec[pl.ds(my * CAP, CAP), :] = bucket_partial(my)

        for off in range(1, N_DEV):
            src = (my + off) % N_DEV
            recv_wait_rows(xr, rx, src)
            recv_wait(ar, ra, src)
            pbuf[off - 1] = bucket_partial(src)
            cp = pltpu.make_async_remote_copy(
                src_ref=pbuf.at[off - 1],
                dst_ref=rrec.at[pl.ds(my * CAP, CAP)],
                send_sem=sp.at[off], recv_sem=rp.at[my],
                device_id=(src,), device_id_type=pl.DeviceIdType.MESH,
            )
            cp.start()
            sends.append(cp)

        for off in range(1, N_DEV):
            s = (my + off) % N_DEV
            recv_wait_rows(rrec, rp, s)
        out_ref[:] = jnp.dot(
            gt_ref[:], rrec[:], preferred_element_type=jnp.float32
        )

        for c in sends:
            c.wait_send()

    return pl.pallas_call(
        body,
        out_shape=jax.ShapeDtypeStruct((t, d), jnp.float32),
        in_specs=[pl.BlockSpec(memory_space=pltpu.VMEM)] * 6,
        out_specs=pl.BlockSpec(memory_space=pltpu.VMEM),
        scratch_shapes=[
            pltpu.VMEM((NC, d), jnp.bfloat16),
            pltpu.VMEM((NC, d), jnp.bfloat16),
            pltpu.VMEM((N_DEV, CAP, 1), jnp.int32),
            pltpu.VMEM((N_DEV - 1, CAP, d), jnp.bfloat16),
            pltpu.VMEM((NC, d), jnp.bfloat16),
            pltpu.SemaphoreType.DMA((N_DEV,)),
            pltpu.SemaphoreType.DMA((N_DEV,)),
            pltpu.SemaphoreType.DMA((N_DEV,)),
            pltpu.SemaphoreType.DMA((N_DEV,)),
            pltpu.SemaphoreType.DMA((N_DEV,)),
            pltpu.SemaphoreType.DMA((N_DEV,)),
        ],
        compiler_params=pltpu.CompilerParams(
            collective_id=0,
            vmem_limit_bytes=48 * 1024 * 1024,
        ),
    )(xb, G, Gt, asend, w1b, w2b)


# device time: 107410 ns/iter; 1.3985x vs baseline; 1.0060x over previous
import jax
import jax.numpy as jnp
from jax import lax
from jax.experimental import pallas as pl
from jax.experimental.pallas import tpu as pltpu

N_DEV = 4
N_LOC_E = 2
CAP = 384
NC = N_DEV * CAP


def kernel(x, assign, W1, W2):
    t, d = x.shape
    _, _, f = W1.shape
    xb = x.astype(jnp.bfloat16)
    w1b = W1.astype(jnp.bfloat16)
    w2b = W2.astype(jnp.bfloat16)

    dest = assign // N_LOC_E
    onehot = (dest[:, None] == jnp.arange(N_DEV)[None, :]).astype(jnp.int32)
    pos = jnp.sum(onehot * jnp.cumsum(onehot, axis=0), axis=1) - 1
    k = dest * CAP + pos
    iota = jnp.arange(NC)
    G = (k[None, :] == iota[:, None]).astype(jnp.bfloat16)
    Gt = (k[:, None] == iota[None, :]).astype(jnp.bfloat16)
    asend = (
        jnp.dot(G.astype(jnp.float32), (assign + 1).astype(jnp.float32))
        .astype(jnp.int32) - 1
    ).reshape(N_DEV, CAP, 1)

    def body(x_ref, g_ref, gt_ref, as_ref, w1_ref, w2_ref, out_ref,
             xsend, xr, ar, pbuf, rrec,
             sx, rx, sa, ra, sp, rp):
        my = lax.axis_index("i")

        barrier = pltpu.get_barrier_semaphore()
        for off in range(1, N_DEV):
            pl.semaphore_signal(
                barrier, inc=1,
                device_id=((my + off) % N_DEV,),
                device_id_type=pl.DeviceIdType.MESH,
            )
        pl.semaphore_wait(barrier, N_DEV - 1)

        def dispatch(dst):
            return jnp.dot(
                g_ref[pl.ds(dst * CAP, CAP), :], x_ref[:],
                preferred_element_type=jnp.float32,
            ).astype(jnp.bfloat16)

        sends = []
        for off in range(1, N_DEV):
            dst = (my + off) % N_DEV
            xsend[pl.ds(dst * CAP, CAP), :] = dispatch(dst)
            cx = pltpu.make_async_remote_copy(
                src_ref=xsend.at[pl.ds(dst * CAP, CAP)],
                dst_ref=xr.at[pl.ds(my * CAP, CAP)],
                send_sem=sx.at[off], recv_sem=rx.at[my],
                device_id=(dst,), device_id_type=pl.DeviceIdType.MESH,
            )
            cx.start()
            ca = pltpu.make_async_remote_copy(
                src_ref=as_ref.at[dst], dst_ref=ar.at[my],
                send_sem=sa.at[off], recv_sem=ra.at[my],
                device_id=(dst,), device_id_type=pl.DeviceIdType.MESH,
            )
            ca.start()
            sends += [cx, ca]
        xr[pl.ds(my * CAP, CAP), :] = dispatch(my)
        ar[my] = as_ref[my]

        def recv_wait_rows(buf, sems, s):
            rcv = pltpu.make_async_remote_copy(
                src_ref=buf.at[pl.ds(s * CAP, CAP)],
                dst_ref=buf.at[pl.ds(s * CAP, CAP)],
                send_sem=sems.at[s], recv_sem=sems.at[s],
                device_id=(my,), device_id_type=pl.DeviceIdType.MESH,
            )
            rcv.wait_recv()

        def recv_wait(buf, sems, s):
            rcv = pltpu.make_async_remote_copy(
                src_ref=buf.at[s], dst_ref=buf.at[s],
                send_sem=sems.at[s], recv_sem=sems.at[s],
                device_id=(my,), device_id_type=pl.DeviceIdType.MESH,
            )
            rcv.wait_recv()

        def bucket_partial(src):
            xs = xr[pl.ds(src * CAP, CAP), :]
            asg = ar[src]
            acc = None
            for le in range(N_LOC_E):
                e = my * N_LOC_E + le
                xm = jnp.where(asg == e, xs, jnp.bfloat16(0.0))
                h = jnp.dot(xm, w1_ref[le], preferred_element_type=jnp.float32)
                h = jnp.maximum(h, 0.0).astype(jnp.bfloat16)
                p = jnp.dot(h, w2_ref[le], preferred_element_type=jnp.float32)
                acc = p if acc is None else acc + p
            return acc.astype(jnp.bfloat16)

        own = bucket_partial(my)
        rrec[pl.ds(my * CAP, CAP), :] = own
        out_ref[:] = jnp.dot(
            gt_ref[:, pl.ds(my * CAP, CAP)], own,
            preferred_element_type=jnp.float32,
        )

        for off in range(1, N_DEV):
            src = (my + off) % N_DEV
            recv_wait_rows(xr, rx, src)
            recv_wait(ar, ra, src)
            pbuf[off - 1] = bucket_partial(src)
            cp = pltpu.make_async_remote_copy(
                src_ref=pbuf.at[off - 1],
                dst_ref=rrec.at[pl.ds(my * CAP, CAP)],
                send_sem=sp.at[off], recv_sem=rp.at[my],
                device_id=(src,), device_id_type=pl.DeviceIdType.MESH,
            )
            cp.start()
            sends.append(cp)

        for off in range(1, N_DEV):
            s = (my + off) % N_DEV
            recv_wait_rows(rrec, rp, s)
            out_ref[:] = out_ref[:] + jnp.dot(
                gt_ref[:, pl.ds(s * CAP, CAP)], rrec[pl.ds(s * CAP, CAP), :],
                preferred_element_type=jnp.float32,
            )

        for c in sends:
            c.wait_send()

    return pl.pallas_call(
        body,
        out_shape=jax.ShapeDtypeStruct((t, d), jnp.float32),
        in_specs=[pl.BlockSpec(memory_space=pltpu.VMEM)] * 6,
        out_specs=pl.BlockSpec(memory_space=pltpu.VMEM),
        scratch_shapes=[
            pltpu.VMEM((NC, d), jnp.bfloat16),
            pltpu.VMEM((NC, d), jnp.bfloat16),
            pltpu.VMEM((N_DEV, CAP, 1), jnp.int32),
            pltpu.VMEM((N_DEV - 1, CAP, d), jnp.bfloat16),
            pltpu.VMEM((NC, d), jnp.bfloat16),
            pltpu.SemaphoreType.DMA((N_DEV,)),
            pltpu.SemaphoreType.DMA((N_DEV,)),
            pltpu.SemaphoreType.DMA((N_DEV,)),
            pltpu.SemaphoreType.DMA((N_DEV,)),
            pltpu.SemaphoreType.DMA((N_DEV,)),
            pltpu.SemaphoreType.DMA((N_DEV,)),
        ],
        compiler_params=pltpu.CompilerParams(
            collective_id=0,
            vmem_limit_bytes=48 * 1024 * 1024,
        ),
    )(xb, G, Gt, asend, w1b, w2b)


# device time: 95898 ns/iter; 1.5663x vs baseline; 1.1200x over previous
import jax
import jax.numpy as jnp
from jax import lax
from jax.experimental import pallas as pl
from jax.experimental.pallas import tpu as pltpu

N_DEV = 4
N_LOC_E = 2
CAP = 384
NC = N_DEV * CAP


def kernel(x, assign, W1, W2):
    t, d = x.shape
    _, _, f = W1.shape
    xb = x.astype(jnp.bfloat16)
    w1b = W1.astype(jnp.bfloat16)
    w2b = W2.astype(jnp.bfloat16)

    dest = assign // N_LOC_E
    onehot = (dest[:, None] == jnp.arange(N_DEV)[None, :]).astype(jnp.int32)
    pos = jnp.sum(onehot * jnp.cumsum(onehot, axis=0), axis=1) - 1
    k = dest * CAP + pos
    iota = jnp.arange(NC)
    G = (k[None, :] == iota[:, None]).astype(jnp.bfloat16)
    Gt = (k[:, None] == iota[None, :]).astype(jnp.bfloat16)
    asend = (
        jnp.dot(G.astype(jnp.float32), (assign + 1).astype(jnp.float32))
        .astype(jnp.int32) - 1
    ).reshape(N_DEV, CAP, 1)

    def body(x_ref, g_ref, gt_ref, as_ref, w1_ref, w2_ref, out_ref,
             xsend, xr, ar, pbuf, rrec,
             sx, rx, sa, ra, sp, rp):
        my = lax.axis_index("i")

        barrier = pltpu.get_barrier_semaphore()
        for off in range(1, N_DEV):
            pl.semaphore_signal(
                barrier, inc=1,
                device_id=((my + off) % N_DEV,),
                device_id_type=pl.DeviceIdType.MESH,
            )
        pl.semaphore_wait(barrier, N_DEV - 1)

        def dispatch(dst):
            return jnp.dot(
                g_ref[pl.ds(dst * CAP, CAP), :], x_ref[:],
                preferred_element_type=jnp.float32,
            ).astype(jnp.bfloat16)

        sends = []
        for off in range(1, N_DEV):
            dst = (my + off) % N_DEV
            xsend[pl.ds(dst * CAP, CAP), :] = dispatch(dst)
            cx = pltpu.make_async_remote_copy(
                src_ref=xsend.at[pl.ds(dst * CAP, CAP)],
                dst_ref=xr.at[pl.ds(my * CAP, CAP)],
                send_sem=sx.at[off], recv_sem=rx.at[my],
                device_id=(dst,), device_id_type=pl.DeviceIdType.MESH,
            )
            cx.start()
            ca = pltpu.make_async_remote_copy(
                src_ref=as_ref.at[dst], dst_ref=ar.at[my],
                send_sem=sa.at[off], recv_sem=ra.at[my],
                device_id=(dst,), device_id_type=pl.DeviceIdType.MESH,
            )
            ca.start()
            sends += [cx, ca]
        xr[pl.ds(my * CAP, CAP), :] = dispatch(my)
        ar[my] = as_ref[my]

        def recv_wait_rows(buf, sems, s):
            rcv = pltpu.make_async_remote_copy(
                src_ref=buf.at[pl.ds(s * CAP, CAP)],
                dst_ref=buf.at[pl.ds(s * CAP, CAP)],
                send_sem=sems.at[s], recv_sem=sems.at[s],
                device_id=(my,), device_id_type=pl.DeviceIdType.MESH,
            )
            rcv.wait_recv()

        def recv_wait(buf, sems, s):
            rcv = pltpu.make_async_remote_copy(
                src_ref=buf.at[s], dst_ref=buf.at[s],
                send_sem=sems.at[s], recv_sem=sems.at[s],
                device_id=(my,), device_id_type=pl.DeviceIdType.MESH,
            )
            rcv.wait_recv()

        def bucket_partial(src):
            xs = xr[pl.ds(src * CAP, CAP), :]
            asg = ar[src]
            acc = None
            for le in range(N_LOC_E):
                e = my * N_LOC_E + le
                xm = jnp.where(asg == e, xs, jnp.bfloat16(0.0))
                h = jnp.dot(xm, w1_ref[le], preferred_element_type=jnp.float32)
                h = jnp.maximum(h, 0.0).astype(jnp.bfloat16)
                p = jnp.dot(h, w2_ref[le], preferred_element_type=jnp.float32)
                acc = p if acc is None else acc + p
            return acc.astype(jnp.bfloat16)

        own = bucket_partial(my)
        rrec[pl.ds(my * CAP, CAP), :] = own
        out_ref[:] = jnp.dot(
            gt_ref[:, pl.ds(my * CAP, CAP)], own,
            preferred_element_type=jnp.float32,
        )

        for off in range(N_DEV - 1, 0, -1):
            src = (my + off) % N_DEV
            recv_wait_rows(xr, rx, src)
            recv_wait(ar, ra, src)
            pbuf[off - 1] = bucket_partial(src)
            cp = pltpu.make_async_remote_copy(
                src_ref=pbuf.at[off - 1],
                dst_ref=rrec.at[pl.ds(my * CAP, CAP)],
                send_sem=sp.at[off], recv_sem=rp.at[my],
                device_id=(src,), device_id_type=pl.DeviceIdType.MESH,
            )
            cp.start()
            sends.append(cp)

        for off in range(1, N_DEV):
            s = (my + off) % N_DEV
            recv_wait_rows(rrec, rp, s)
            out_ref[:] = out_ref[:] + jnp.dot(
                gt_ref[:, pl.ds(s * CAP, CAP)], rrec[pl.ds(s * CAP, CAP), :],
                preferred_element_type=jnp.float32,
            )

        for c in sends:
            c.wait_send()

    return pl.pallas_call(
        body,
        out_shape=jax.ShapeDtypeStruct((t, d), jnp.float32),
        in_specs=[pl.BlockSpec(memory_space=pltpu.VMEM)] * 6,
        out_specs=pl.BlockSpec(memory_space=pltpu.VMEM),
        scratch_shapes=[
            pltpu.VMEM((NC, d), jnp.bfloat16),
            pltpu.VMEM((NC, d), jnp.bfloat16),
            pltpu.VMEM((N_DEV, CAP, 1), jnp.int32),
            pltpu.VMEM((N_DEV - 1, CAP, d), jnp.bfloat16),
            pltpu.VMEM((NC, d), jnp.bfloat16),
            pltpu.SemaphoreType.DMA((N_DEV,)),
            pltpu.SemaphoreType.DMA((N_DEV,)),
            pltpu.SemaphoreType.DMA((N_DEV,)),
            pltpu.SemaphoreType.DMA((N_DEV,)),
            pltpu.SemaphoreType.DMA((N_DEV,)),
            pltpu.SemaphoreType.DMA((N_DEV,)),
        ],
        compiler_params=pltpu.CompilerParams(
            collective_id=0,
            vmem_limit_bytes=48 * 1024 * 1024,
        ),
    )(xb, G, Gt, asend, w1b, w2b)


# device time: 88073 ns/iter; 1.7055x vs baseline; 1.0888x over previous
import jax
import jax.numpy as jnp
from jax import lax
from jax.experimental import pallas as pl
from jax.experimental.pallas import tpu as pltpu

N_DEV = 4
N_LOC_E = 2
CAP = 384
NC = N_DEV * CAP


def kernel(x, assign, W1, W2):
    t, d = x.shape
    _, _, f = W1.shape
    xb = x.astype(jnp.bfloat16)
    w1b = W1.astype(jnp.bfloat16)
    w2b = W2.astype(jnp.bfloat16)

    dest = assign // N_LOC_E
    onehot = (dest[:, None] == jnp.arange(N_DEV)[None, :]).astype(jnp.int32)
    pos = jnp.sum(onehot * jnp.cumsum(onehot, axis=0), axis=1) - 1
    k = dest * CAP + pos
    krow = k.reshape(1, t)
    kcol = k.reshape(t, 1)
    a1 = (assign + 1).astype(jnp.bfloat16).reshape(t, 1)

    def body(x_ref, kr_ref, kc_ref, a1_ref, w1_ref, w2_ref, out_ref,
             xsend, xr, ar, asv, pbuf, rrec,
             sx, rx, sa, ra, sp, rp):
        my = lax.axis_index("i")

        barrier = pltpu.get_barrier_semaphore()
        for off in range(1, N_DEV):
            pl.semaphore_signal(
                barrier, inc=1,
                device_id=((my + off) % N_DEV,),
                device_id_type=pl.DeviceIdType.MESH,
            )
        pl.semaphore_wait(barrier, N_DEV - 1)

        def dispatch(dst):
            rows = lax.broadcasted_iota(jnp.int32, (CAP, t), 0) + dst * CAP
            gm = (rows == kr_ref[:]).astype(jnp.bfloat16)
            xsl = jnp.dot(
                gm, x_ref[:], preferred_element_type=jnp.float32
            ).astype(jnp.bfloat16)
            asl = jnp.dot(
                gm, a1_ref[:], preferred_element_type=jnp.float32
            ).astype(jnp.bfloat16)
            return xsl, asl

        sends = []
        for off in range(1, N_DEV):
            dst = (my + off) % N_DEV
            xsl, asl = dispatch(dst)
            xsend[pl.ds(dst * CAP, CAP), :] = xsl
            asv[dst] = asl
            cx = pltpu.make_async_remote_copy(
                src_ref=xsend.at[pl.ds(dst * CAP, CAP)],
                dst_ref=xr.at[pl.ds(my * CAP, CAP)],
                send_sem=sx.at[off], recv_sem=rx.at[my],
                device_id=(dst,), device_id_type=pl.DeviceIdType.MESH,
            )
            cx.start()
            ca = pltpu.make_async_remote_copy(
                src_ref=asv.at[dst], dst_ref=ar.at[my],
                send_sem=sa.at[off], recv_sem=ra.at[my],
                device_id=(dst,), device_id_type=pl.DeviceIdType.MESH,
            )
            ca.start()
            sends += [cx, ca]
        xsl, asl = dispatch(my)
        xr[pl.ds(my * CAP, CAP), :] = xsl
        ar[my] = asl

        def recv_wait_rows(buf, sems, s):
            rcv = pltpu.make_async_remote_copy(
                src_ref=buf.at[pl.ds(s * CAP, CAP)],
                dst_ref=buf.at[pl.ds(s * CAP, CAP)],
                send_sem=sems.at[s], recv_sem=sems.at[s],
                device_id=(my,), device_id_type=pl.DeviceIdType.MESH,
            )
            rcv.wait_recv()

        def recv_wait(buf, sems, s):
            rcv = pltpu.make_async_remote_copy(
                src_ref=buf.at[s], dst_ref=buf.at[s],
                send_sem=sems.at[s], recv_sem=sems.at[s],
                device_id=(my,), device_id_type=pl.DeviceIdType.MESH,
            )
            rcv.wait_recv()

        def bucket_partial(src):
            xs = xr[pl.ds(src * CAP, CAP), :]
            asg = ar[src]
            acc = None
            for le in range(N_LOC_E):
                ev = (my * N_LOC_E + le + 1).astype(jnp.bfloat16)
                xm = jnp.where(asg == ev, xs, jnp.bfloat16(0.0))
                h = jnp.dot(xm, w1_ref[le], preferred_element_type=jnp.float32)
                h = jnp.maximum(h, 0.0).astype(jnp.bfloat16)
                p = jnp.dot(h, w2_ref[le], preferred_element_type=jnp.float32)
                acc = p if acc is None else acc + p
            return acc.astype(jnp.bfloat16)

        def gt_slice(s):
            cols = lax.broadcasted_iota(jnp.int32, (t, CAP), 1) + s * CAP
            return (cols == kc_ref[:]).astype(jnp.bfloat16)

        own = bucket_partial(my)
        rrec[pl.ds(my * CAP, CAP), :] = own
        out_ref[:] = jnp.dot(
            gt_slice(my), own, preferred_element_type=jnp.float32
        )

        for off in range(N_DEV - 1, 0, -1):
            src = (my + off) % N_DEV
            recv_wait_rows(xr, rx, src)
            recv_wait(ar, ra, src)
            pbuf[off - 1] = bucket_partial(src)
            cp = pltpu.make_async_remote_copy(
                src_ref=pbuf.at[off - 1],
                dst_ref=rrec.at[pl.ds(my * CAP, CAP)],
                send_sem=sp.at[off], recv_sem=rp.at[my],
                device_id=(src,), device_id_type=pl.DeviceIdType.MESH,
            )
            cp.start()
            sends.append(cp)

        for off in range(1, N_DEV):
            s = (my + off) % N_DEV
            recv_wait_rows(rrec, rp, s)
            out_ref[:] = out_ref[:] + jnp.dot(
                gt_slice(s), rrec[pl.ds(s * CAP, CAP), :],
                preferred_element_type=jnp.float32,
            )

        for c in sends:
            c.wait_send()

    return pl.pallas_call(
        body,
        out_shape=jax.ShapeDtypeStruct((t, d), jnp.float32),
        in_specs=[pl.BlockSpec(memory_space=pltpu.VMEM)] * 6,
        out_specs=pl.BlockSpec(memory_space=pltpu.VMEM),
        scratch_shapes=[
            pltpu.VMEM((NC, d), jnp.bfloat16),
            pltpu.VMEM((NC, d), jnp.bfloat16),
            pltpu.VMEM((N_DEV, CAP, 1), jnp.bfloat16),
            pltpu.VMEM((N_DEV, CAP, 1), jnp.bfloat16),
            pltpu.VMEM((N_DEV - 1, CAP, d), jnp.bfloat16),
            pltpu.VMEM((NC, d), jnp.bfloat16),
            pltpu.SemaphoreType.DMA((N_DEV,)),
            pltpu.SemaphoreType.DMA((N_DEV,)),
            pltpu.SemaphoreType.DMA((N_DEV,)),
            pltpu.SemaphoreType.DMA((N_DEV,)),
            pltpu.SemaphoreType.DMA((N_DEV,)),
            pltpu.SemaphoreType.DMA((N_DEV,)),
        ],
        compiler_params=pltpu.CompilerParams(
            collective_id=0,
            vmem_limit_bytes=48 * 1024 * 1024,
        ),
    )(xb, krow, kcol, a1, w1b, w2b)


# device time: 85836 ns/iter; 1.7500x vs baseline; 1.0261x over previous
import jax
import jax.numpy as jnp
from jax import lax
from jax.experimental import pallas as pl
from jax.experimental.pallas import tpu as pltpu

N_DEV = 4
N_LOC_E = 2
CAP = 384
NC = N_DEV * CAP


def kernel(x, assign, W1, W2):
    t, d = x.shape
    _, _, f = W1.shape
    xb = x.astype(jnp.bfloat16)

    dest = assign // N_LOC_E
    onehot = (dest[:, None] == jnp.arange(N_DEV)[None, :]).astype(jnp.int32)
    pos = jnp.sum(onehot * jnp.cumsum(onehot, axis=0), axis=1) - 1
    k = dest * CAP + pos
    krow = k.reshape(1, t)
    kcol = k.reshape(t, 1)
    a1 = (assign + 1).astype(jnp.bfloat16).reshape(t, 1)

    WR = 256

    def body(x_ref, kr_ref, kc_ref, a1_ref, w1_ref, w2_ref, out_ref,
             xsend, xr, ar, asv, pbuf, rrec, w1b, w2b, wst,
             sx, rx, sa, ra, sp, rp, wsem):
        my = lax.axis_index("i")

        chunks = []
        for le in range(N_LOC_E):
            for r in range(0, d, WR):
                chunks.append(("w1", le, r))
            for r in range(0, f, WR):
                chunks.append(("w2", le, r))

        def start_copy(i, slot):
            kind, le, r = chunks[i]
            if kind == "w1":
                c = pltpu.make_async_copy(
                    w1_ref.at[le, pl.ds(r, WR), :],
                    wst.at[slot], wsem.at[slot])
            else:
                c = pltpu.make_async_copy(
                    w2_ref.at[le, pl.ds(r, WR), :],
                    wst.at[slot, :, pl.ds(0, d)], wsem.at[slot])
            c.start()
            return c

        copies = [None] * len(chunks)
        copies[0] = start_copy(0, 0)
        copies[1] = start_copy(1, 1)

        def convert(i):
            kind, le, r = chunks[i]
            slot = i % 2
            copies[i].wait()
            if kind == "w1":
                w1b[le, pl.ds(r, WR), :] = wst[slot].astype(jnp.bfloat16)
            else:
                w2b[le, pl.ds(r, WR), :] = (
                    wst[slot, :, pl.ds(0, d)].astype(jnp.bfloat16))
            if i + 2 < len(chunks):
                copies[i + 2] = start_copy(i + 2, slot)

        barrier = pltpu.get_barrier_semaphore()
        for off in range(1, N_DEV):
            pl.semaphore_signal(
                barrier, inc=1,
                device_id=((my + off) % N_DEV,),
                device_id_type=pl.DeviceIdType.MESH,
            )
        pl.semaphore_wait(barrier, N_DEV - 1)

        def dispatch(dst):
            rows = lax.broadcasted_iota(jnp.int32, (CAP, t), 0) + dst * CAP
            gm = (rows == kr_ref[:]).astype(jnp.bfloat16)
            xsl = jnp.dot(
                gm, x_ref[:], preferred_element_type=jnp.float32
            ).astype(jnp.bfloat16)
            asl = jnp.dot(
                gm, a1_ref[:], preferred_element_type=jnp.float32
            ).astype(jnp.bfloat16)
            return xsl, asl

        sends = []
        for off in range(1, N_DEV):
            dst = (my + off) % N_DEV
            xsl, asl = dispatch(dst)
            xsend[pl.ds(dst * CAP, CAP), :] = xsl
            asv[dst] = asl
            cx = pltpu.make_async_remote_copy(
                src_ref=xsend.at[pl.ds(dst * CAP, CAP)],
                dst_ref=xr.at[pl.ds(my * CAP, CAP)],
                send_sem=sx.at[off], recv_sem=rx.at[my],
                device_id=(dst,), device_id_type=pl.DeviceIdType.MESH,
            )
            cx.start()
            ca = pltpu.make_async_remote_copy(
                src_ref=asv.at[dst], dst_ref=ar.at[my],
                send_sem=sa.at[off], recv_sem=ra.at[my],
                device_id=(dst,), device_id_type=pl.DeviceIdType.MESH,
            )
            ca.start()
            sends += [cx, ca]
        xsl, asl = dispatch(my)
        xr[pl.ds(my * CAP, CAP), :] = xsl
        ar[my] = asl

        for i in range(len(chunks)):
            convert(i)

        def recv_wait_rows(buf, sems, s):
            rcv = pltpu.make_async_remote_copy(
                src_ref=buf.at[pl.ds(s * CAP, CAP)],
                dst_ref=buf.at[pl.ds(s * CAP, CAP)],
                send_sem=sems.at[s], recv_sem=sems.at[s],
                device_id=(my,), device_id_type=pl.DeviceIdType.MESH,
            )
            rcv.wait_recv()

        def recv_wait(buf, sems, s):
            rcv = pltpu.make_async_remote_copy(
                src_ref=buf.at[s], dst_ref=buf.at[s],
                send_sem=sems.at[s], recv_sem=sems.at[s],
                device_id=(my,), device_id_type=pl.DeviceIdType.MESH,
            )
            rcv.wait_recv()

        def bucket_partial(src):
            xs = xr[pl.ds(src * CAP, CAP), :]
            asg = ar[src]
            acc = None
            for le in range(N_LOC_E):
                ev = (my * N_LOC_E + le + 1).astype(jnp.bfloat16)
                xm = jnp.where(asg == ev, xs, jnp.bfloat16(0.0))
                h = jnp.dot(xm, w1b[le], preferred_element_type=jnp.float32)
                h = jnp.maximum(h, 0.0).astype(jnp.bfloat16)
                p = jnp.dot(h, w2b[le], preferred_element_type=jnp.float32)
                acc = p if acc is None else acc + p
            return acc.astype(jnp.bfloat16)

        def gt_slice(s):
            cols = lax.broadcasted_iota(jnp.int32, (t, CAP), 1) + s * CAP
            return (cols == kc_ref[:]).astype(jnp.bfloat16)

        own = bucket_partial(my)
        rrec[pl.ds(my * CAP, CAP), :] = own
        out_ref[:] = jnp.dot(
            gt_slice(my), own, preferred_element_type=jnp.float32
        )

        for off in range(N_DEV - 1, 0, -1):
            src = (my + off) % N_DEV
            recv_wait_rows(xr, rx, src)
            recv_wait(ar, ra, src)
            pbuf[off - 1] = bucket_partial(src)
            cp = pltpu.make_async_remote_copy(
                src_ref=pbuf.at[off - 1],
                dst_ref=rrec.at[pl.ds(my * CAP, CAP)],
                send_sem=sp.at[off], recv_sem=rp.at[my],
                device_id=(src,), device_id_type=pl.DeviceIdType.MESH,
            )
            cp.start()
            sends.append(cp)

        for off in range(1, N_DEV):
            s = (my + off) % N_DEV
            recv_wait_rows(rrec, rp, s)
            out_ref[:] = out_ref[:] + jnp.dot(
                gt_slice(s), rrec[pl.ds(s * CAP, CAP), :],
                preferred_element_type=jnp.float32,
            )

        for c in sends:
            c.wait_send()

    return pl.pallas_call(
        body,
        out_shape=jax.ShapeDtypeStruct((t, d), jnp.float32),
        in_specs=(
            [pl.BlockSpec(memory_space=pltpu.VMEM)] * 4
            + [pl.BlockSpec(memory_space=pl.ANY)] * 2
        ),
        out_specs=pl.BlockSpec(memory_space=pltpu.VMEM),
        scratch_shapes=[
            pltpu.VMEM((NC, d), jnp.bfloat16),
            pltpu.VMEM((NC, d), jnp.bfloat16),
            pltpu.VMEM((N_DEV, CAP, 1), jnp.bfloat16),
            pltpu.VMEM((N_DEV, CAP, 1), jnp.bfloat16),
            pltpu.VMEM((N_DEV - 1, CAP, d), jnp.bfloat16),
            pltpu.VMEM((NC, d), jnp.bfloat16),
            pltpu.VMEM((N_LOC_E, d, f), jnp.bfloat16),
            pltpu.VMEM((N_LOC_E, f, d), jnp.bfloat16),
            pltpu.VMEM((2, WR, f), jnp.float32),
            pltpu.SemaphoreType.DMA((N_DEV,)),
            pltpu.SemaphoreType.DMA((N_DEV,)),
            pltpu.SemaphoreType.DMA((N_DEV,)),
            pltpu.SemaphoreType.DMA((N_DEV,)),
            pltpu.SemaphoreType.DMA((N_DEV,)),
            pltpu.SemaphoreType.DMA((N_DEV,)),
            pltpu.SemaphoreType.DMA((2,)),
        ],
        compiler_params=pltpu.CompilerParams(
            collective_id=0,
            vmem_limit_bytes=56 * 1024 * 1024,
        ),
    )(xb, krow, kcol, a1, W1, W2)


# device time: 78984 ns/iter; 1.9018x vs baseline; 1.0868x over previous
import jax
import jax.numpy as jnp
from jax import lax
from jax.experimental import pallas as pl
from jax.experimental.pallas import tpu as pltpu

N_DEV = 4
N_LOC_E = 2
CAP = 320
NC = N_DEV * CAP


def kernel(x, assign, W1, W2):
    t, d = x.shape
    _, _, f = W1.shape
    xb = x.astype(jnp.bfloat16)

    dest = assign // N_LOC_E
    onehot = (dest[:, None] == jnp.arange(N_DEV)[None, :]).astype(jnp.int32)
    pos = jnp.sum(onehot * jnp.cumsum(onehot, axis=0), axis=1) - 1
    k = dest * CAP + pos
    krow = k.reshape(1, t)
    kcol = k.reshape(t, 1)
    a1 = (assign + 1).astype(jnp.bfloat16).reshape(t, 1)

    WR = 256

    def body(x_ref, kr_ref, kc_ref, a1_ref, w1_ref, w2_ref, out_ref,
             xsend, xr, ar, asv, pbuf, rrec, w1b, w2b, wst,
             sx, rx, sa, ra, sp, rp, wsem):
        my = lax.axis_index("i")

        chunks = []
        for le in range(N_LOC_E):
            for r in range(0, d, WR):
                chunks.append(("w1", le, r))
            for r in range(0, f, WR):
                chunks.append(("w2", le, r))

        def start_copy(i, slot):
            kind, le, r = chunks[i]
            if kind == "w1":
                c = pltpu.make_async_copy(
                    w1_ref.at[le, pl.ds(r, WR), :],
                    wst.at[slot], wsem.at[slot])
            else:
                c = pltpu.make_async_copy(
                    w2_ref.at[le, pl.ds(r, WR), :],
                    wst.at[slot, :, pl.ds(0, d)], wsem.at[slot])
            c.start()
            return c

        copies = [None] * len(chunks)
        copies[0] = start_copy(0, 0)
        copies[1] = start_copy(1, 1)

        def convert(i):
            kind, le, r = chunks[i]
            slot = i % 2
            copies[i].wait()
            if kind == "w1":
                w1b[le, pl.ds(r, WR), :] = wst[slot].astype(jnp.bfloat16)
            else:
                w2b[le, pl.ds(r, WR), :] = (
                    wst[slot, :, pl.ds(0, d)].astype(jnp.bfloat16))
            if i + 2 < len(chunks):
                copies[i + 2] = start_copy(i + 2, slot)

        barrier = pltpu.get_barrier_semaphore()
        for off in range(1, N_DEV):
            pl.semaphore_signal(
                barrier, inc=1,
                device_id=((my + off) % N_DEV,),
                device_id_type=pl.DeviceIdType.MESH,
            )
        pl.semaphore_wait(barrier, N_DEV - 1)

        def dispatch(dst):
            rows = lax.broadcasted_iota(jnp.int32, (CAP, t), 0) + dst * CAP
            gm = (rows == kr_ref[:]).astype(jnp.bfloat16)
            xsl = jnp.dot(
                gm, x_ref[:], preferred_element_type=jnp.float32
            ).astype(jnp.bfloat16)
            asl = jnp.dot(
                gm, a1_ref[:], preferred_element_type=jnp.float32
            ).astype(jnp.bfloat16)
            return xsl, asl

        sends = []
        for off in range(1, N_DEV):
            dst = (my + off) % N_DEV
            xsl, asl = dispatch(dst)
            xsend[pl.ds(dst * CAP, CAP), :] = xsl
            asv[dst] = asl
            cx = pltpu.make_async_remote_copy(
                src_ref=xsend.at[pl.ds(dst * CAP, CAP)],
                dst_ref=xr.at[pl.ds(my * CAP, CAP)],
                send_sem=sx.at[off], recv_sem=rx.at[my],
                device_id=(dst,), device_id_type=pl.DeviceIdType.MESH,
            )
            cx.start()
            ca = pltpu.make_async_remote_copy(
                src_ref=asv.at[dst], dst_ref=ar.at[my],
                send_sem=sa.at[off], recv_sem=ra.at[my],
                device_id=(dst,), device_id_type=pl.DeviceIdType.MESH,
            )
            ca.start()
            sends += [cx, ca]
        xsl, asl = dispatch(my)
        xr[pl.ds(my * CAP, CAP), :] = xsl
        ar[my] = asl

        for i in range(len(chunks)):
            convert(i)

        def recv_wait_rows(buf, sems, s):
            rcv = pltpu.make_async_remote_copy(
                src_ref=buf.at[pl.ds(s * CAP, CAP)],
                dst_ref=buf.at[pl.ds(s * CAP, CAP)],
                send_sem=sems.at[s], recv_sem=sems.at[s],
                device_id=(my,), device_id_type=pl.DeviceIdType.MESH,
            )
            rcv.wait_recv()

        def recv_wait(buf, sems, s):
            rcv = pltpu.make_async_remote_copy(
                src_ref=buf.at[s], dst_ref=buf.at[s],
                send_sem=sems.at[s], recv_sem=sems.at[s],
                device_id=(my,), device_id_type=pl.DeviceIdType.MESH,
            )
            rcv.wait_recv()

        def bucket_partial(src):
            xs = xr[pl.ds(src * CAP, CAP), :]
            asg = ar[src]
            acc = None
            for le in range(N_LOC_E):
                ev = (my * N_LOC_E + le + 1).astype(jnp.bfloat16)
                xm = jnp.where(asg == ev, xs, jnp.bfloat16(0.0))
                h = jnp.dot(xm, w1b[le], preferred_element_type=jnp.float32)
                h = jnp.maximum(h, 0.0).astype(jnp.bfloat16)
                p = jnp.dot(h, w2b[le], preferred_element_type=jnp.float32)
                acc = p if acc is None else acc + p
            return acc.astype(jnp.bfloat16)

        def gt_slice(s):
            cols = lax.broadcasted_iota(jnp.int32, (t, CAP), 1) + s * CAP
            return (cols == kc_ref[:]).astype(jnp.bfloat16)

        own = bucket_partial(my)
        rrec[pl.ds(my * CAP, CAP), :] = own
        out_ref[:] = jnp.dot(
            gt_slice(my), own, preferred_element_type=jnp.float32
        )

        for off in range(N_DEV - 1, 0, -1):
            src = (my + off) % N_DEV
            recv_wait_rows(xr, rx, src)
            recv_wait(ar, ra, src)
            pbuf[off - 1] = bucket_partial(src)
            cp = pltpu.make_async_remote_copy(
                src_ref=pbuf.at[off - 1],
                dst_ref=rrec.at[pl.ds(my * CAP, CAP)],
                send_sem=sp.at[off], recv_sem=rp.at[my],
                device_id=(src,), device_id_type=pl.DeviceIdType.MESH,
            )
            cp.start()
            sends.append(cp)

        for off in range(1, N_DEV):
            s = (my + off) % N_DEV
            recv_wait_rows(rrec, rp, s)
            out_ref[:] = out_ref[:] + jnp.dot(
                gt_slice(s), rrec[pl.ds(s * CAP, CAP), :],
                preferred_element_type=jnp.float32,
            )

        for c in sends:
            c.wait_send()

    return pl.pallas_call(
        body,
        out_shape=jax.ShapeDtypeStruct((t, d), jnp.float32),
        in_specs=(
            [pl.BlockSpec(memory_space=pltpu.VMEM)] * 4
            + [pl.BlockSpec(memory_space=pl.ANY)] * 2
        ),
        out_specs=pl.BlockSpec(memory_space=pltpu.VMEM),
        scratch_shapes=[
            pltpu.VMEM((NC, d), jnp.bfloat16),
            pltpu.VMEM((NC, d), jnp.bfloat16),
            pltpu.VMEM((N_DEV, CAP, 1), jnp.bfloat16),
            pltpu.VMEM((N_DEV, CAP, 1), jnp.bfloat16),
            pltpu.VMEM((N_DEV - 1, CAP, d), jnp.bfloat16),
            pltpu.VMEM((NC, d), jnp.bfloat16),
            pltpu.VMEM((N_LOC_E, d, f), jnp.bfloat16),
            pltpu.VMEM((N_LOC_E, f, d), jnp.bfloat16),
            pltpu.VMEM((2, WR, f), jnp.float32),
            pltpu.SemaphoreType.DMA((N_DEV,)),
            pltpu.SemaphoreType.DMA((N_DEV,)),
            pltpu.SemaphoreType.DMA((N_DEV,)),
            pltpu.SemaphoreType.DMA((N_DEV,)),
            pltpu.SemaphoreType.DMA((N_DEV,)),
            pltpu.SemaphoreType.DMA((N_DEV,)),
            pltpu.SemaphoreType.DMA((2,)),
        ],
        compiler_params=pltpu.CompilerParams(
            collective_id=0,
            vmem_limit_bytes=56 * 1024 * 1024,
        ),
    )(xb, krow, kcol, a1, W1, W2)


# device time: 70842 ns/iter; 2.1203x vs baseline; 1.1149x over previous
import jax
import jax.numpy as jnp
from jax import lax
from jax.experimental import pallas as pl
from jax.experimental.pallas import tpu as pltpu

N_DEV = 4
N_LOC_E = 2
CAP = 320
NC = N_DEV * CAP


def kernel(x, assign, W1, W2):
    t, d = x.shape
    _, _, f = W1.shape
    xb = x.astype(jnp.bfloat16)

    dest = assign // N_LOC_E
    onehot = (dest[:, None] == jnp.arange(N_DEV)[None, :]).astype(jnp.int32)
    pos = jnp.sum(onehot * jnp.cumsum(onehot, axis=0), axis=1) - 1
    k = dest * CAP + pos
    krow = k.reshape(1, t)
    kcol = k.reshape(t, 1)
    a1 = (assign + 1).astype(jnp.bfloat16).reshape(t, 1)

    WR = 256

    def body(x_ref, kr_ref, kc_ref, a1_ref, w1_ref, w2_ref, out_ref,
             xsend, xr, ar, asv, pbuf, rrec, w1b, w2b, wst,
             sx, rx, sa, ra, sp, rp, wsem):
        my = lax.axis_index("i")

        chunks = []
        for le in range(N_LOC_E):
            for r in range(0, d, WR):
                chunks.append(("w1", le, r))
            for r in range(0, f, WR):
                chunks.append(("w2", le, r))

        def start_copy(i, slot):
            kind, le, r = chunks[i]
            if kind == "w1":
                c = pltpu.make_async_copy(
                    w1_ref.at[le, pl.ds(r, WR), :],
                    wst.at[slot], wsem.at[slot])
            else:
                c = pltpu.make_async_copy(
                    w2_ref.at[le, pl.ds(r, WR), :],
                    wst.at[slot, :, pl.ds(0, d)], wsem.at[slot])
            c.start()
            return c

        NSLOT = 4
        copies = [None] * len(chunks)
        for i in range(NSLOT):
            copies[i] = start_copy(i, i)

        def convert(i):
            kind, le, r = chunks[i]
            slot = i % NSLOT
            copies[i].wait()
            if kind == "w1":
                w1b[le, pl.ds(r, WR), :] = wst[slot].astype(jnp.bfloat16)
            else:
                w2b[le, pl.ds(r, WR), :] = (
                    wst[slot, :, pl.ds(0, d)].astype(jnp.bfloat16))
            if i + NSLOT < len(chunks):
                copies[i + NSLOT] = start_copy(i + NSLOT, slot)

        barrier = pltpu.get_barrier_semaphore()
        for off in range(1, N_DEV):
            pl.semaphore_signal(
                barrier, inc=1,
                device_id=((my + off) % N_DEV,),
                device_id_type=pl.DeviceIdType.MESH,
            )
        pl.semaphore_wait(barrier, N_DEV - 1)

        def dispatch(dst):
            rows = lax.broadcasted_iota(jnp.int32, (CAP, t), 0) + dst * CAP
            gm = (rows == kr_ref[:]).astype(jnp.bfloat16)
            xsl = jnp.dot(
                gm, x_ref[:], preferred_element_type=jnp.float32
            ).astype(jnp.bfloat16)
            asl = jnp.dot(
                gm, a1_ref[:], preferred_element_type=jnp.float32
            ).astype(jnp.bfloat16)
            return xsl, asl

        sends = []
        for off in range(1, N_DEV):
            dst = (my + off) % N_DEV
            xsl, asl = dispatch(dst)
            xsend[pl.ds(dst * CAP, CAP), :] = xsl
            asv[dst] = asl
            cx = pltpu.make_async_remote_copy(
                src_ref=xsend.at[pl.ds(dst * CAP, CAP)],
                dst_ref=xr.at[pl.ds(my * CAP, CAP)],
                send_sem=sx.at[off], recv_sem=rx.at[my],
                device_id=(dst,), device_id_type=pl.DeviceIdType.MESH,
            )
            cx.start()
            ca = pltpu.make_async_remote_copy(
                src_ref=asv.at[dst], dst_ref=ar.at[my],
                send_sem=sa.at[off], recv_sem=ra.at[my],
                device_id=(dst,), device_id_type=pl.DeviceIdType.MESH,
            )
            ca.start()
            sends += [cx, ca]
        xsl, asl = dispatch(my)
        xr[pl.ds(my * CAP, CAP), :] = xsl
        ar[my] = asl

        def recv_wait_rows(buf, sems, s):
            rcv = pltpu.make_async_remote_copy(
                src_ref=buf.at[pl.ds(s * CAP, CAP)],
                dst_ref=buf.at[pl.ds(s * CAP, CAP)],
                send_sem=sems.at[s], recv_sem=sems.at[s],
                device_id=(my,), device_id_type=pl.DeviceIdType.MESH,
            )
            rcv.wait_recv()

        def recv_wait(buf, sems, s):
            rcv = pltpu.make_async_remote_copy(
                src_ref=buf.at[s], dst_ref=buf.at[s],
                send_sem=sems.at[s], recv_sem=sems.at[s],
                device_id=(my,), device_id_type=pl.DeviceIdType.MESH,
            )
            rcv.wait_recv()

        def expert_contrib(xs, asg, le):
            ev = (my * N_LOC_E + le + 1).astype(jnp.bfloat16)
            xm = jnp.where(asg == ev, xs, jnp.bfloat16(0.0))
            h = jnp.dot(xm, w1b[le], preferred_element_type=jnp.float32)
            h = jnp.maximum(h, 0.0).astype(jnp.bfloat16)
            return jnp.dot(h, w2b[le], preferred_element_type=jnp.float32)

        def bucket_partial(src):
            xs = xr[pl.ds(src * CAP, CAP), :]
            asg = ar[src]
            acc = None
            for le in range(N_LOC_E):
                p = expert_contrib(xs, asg, le)
                acc = p if acc is None else acc + p
            return acc.astype(jnp.bfloat16)

        def gt_slice(s):
            cols = lax.broadcasted_iota(jnp.int32, (t, CAP), 1) + s * CAP
            return (cols == kc_ref[:]).astype(jnp.bfloat16)

        half = len(chunks) // N_LOC_E
        for i in range(half):
            convert(i)
        xs_my = xr[pl.ds(my * CAP, CAP), :]
        asg_my = ar[my]
        own0 = expert_contrib(xs_my, asg_my, 0)
        for i in range(half, len(chunks)):
            convert(i)
        own = (own0 + expert_contrib(xs_my, asg_my, 1)).astype(jnp.bfloat16)
        rrec[pl.ds(my * CAP, CAP), :] = own
        out_ref[:] = jnp.dot(
            gt_slice(my), own, preferred_element_type=jnp.float32
        )

        for off in range(N_DEV - 1, 0, -1):
            src = (my + off) % N_DEV
            recv_wait_rows(xr, rx, src)
            recv_wait(ar, ra, src)
            pbuf[off - 1] = bucket_partial(src)
            cp = pltpu.make_async_remote_copy(
                src_ref=pbuf.at[off - 1],
                dst_ref=rrec.at[pl.ds(my * CAP, CAP)],
                send_sem=sp.at[off], recv_sem=rp.at[my],
                device_id=(src,), device_id_type=pl.DeviceIdType.MESH,
            )
            cp.start()
            sends.append(cp)

        for off in range(1, N_DEV):
            s = (my + off) % N_DEV
            recv_wait_rows(rrec, rp, s)
            out_ref[:] = out_ref[:] + jnp.dot(
                gt_slice(s), rrec[pl.ds(s * CAP, CAP), :],
                preferred_element_type=jnp.float32,
            )

        for c in sends:
            c.wait_send()

    return pl.pallas_call(
        body,
        out_shape=jax.ShapeDtypeStruct((t, d), jnp.float32),
        in_specs=(
            [pl.BlockSpec(memory_space=pltpu.VMEM)] * 4
            + [pl.BlockSpec(memory_space=pl.ANY)] * 2
        ),
        out_specs=pl.BlockSpec(memory_space=pltpu.VMEM),
        scratch_shapes=[
            pltpu.VMEM((NC, d), jnp.bfloat16),
            pltpu.VMEM((NC, d), jnp.bfloat16),
            pltpu.VMEM((N_DEV, CAP, 1), jnp.bfloat16),
            pltpu.VMEM((N_DEV, CAP, 1), jnp.bfloat16),
            pltpu.VMEM((N_DEV - 1, CAP, d), jnp.bfloat16),
            pltpu.VMEM((NC, d), jnp.bfloat16),
            pltpu.VMEM((N_LOC_E, d, f), jnp.bfloat16),
            pltpu.VMEM((N_LOC_E, f, d), jnp.bfloat16),
            pltpu.VMEM((4, WR, f), jnp.float32),
            pltpu.SemaphoreType.DMA((N_DEV,)),
            pltpu.SemaphoreType.DMA((N_DEV,)),
            pltpu.SemaphoreType.DMA((N_DEV,)),
            pltpu.SemaphoreType.DMA((N_DEV,)),
            pltpu.SemaphoreType.DMA((N_DEV,)),
            pltpu.SemaphoreType.DMA((N_DEV,)),
            pltpu.SemaphoreType.DMA((4,)),
        ],
        compiler_params=pltpu.CompilerParams(
            collective_id=0,
            vmem_limit_bytes=56 * 1024 * 1024,
        ),
    )(xb, krow, kcol, a1, W1, W2)


# device time: 63127 ns/iter; 2.3795x vs baseline; 1.1222x over previous
import jax
import jax.numpy as jnp
from jax import lax
from jax.experimental import pallas as pl
from jax.experimental.pallas import tpu as pltpu

N_DEV = 4
N_LOC_E = 2
N_EXP = 8
CAP_E = 176
CAP = N_LOC_E * CAP_E
NC = N_DEV * CAP


def kernel(x, assign, W1, W2):
    t, d = x.shape
    _, _, f = W1.shape
    xb = x.astype(jnp.bfloat16)

    onehot = (assign[:, None] == jnp.arange(N_EXP)[None, :]).astype(jnp.int32)
    pos = jnp.sum(onehot * jnp.cumsum(onehot, axis=0), axis=1) - 1
    k = assign * CAP_E + pos
    krow = k.reshape(1, t)
    kcol = k.reshape(t, 1)

    WR = 256

    def body(x_ref, kr_ref, kc_ref, w1_ref, w2_ref, out_ref,
             xsend, xr, pbuf, rrec, w1b, w2b, wst,
             sx, rx, sp, rp, wsem):
        my = lax.axis_index("i")

        chunks = []
        for le in range(N_LOC_E):
            for r in range(0, d, WR):
                chunks.append(("w1", le, r))
            for r in range(0, f, WR):
                chunks.append(("w2", le, r))

        def start_copy(i, slot):
            kind, le, r = chunks[i]
            if kind == "w1":
                c = pltpu.make_async_copy(
                    w1_ref.at[le, pl.ds(r, WR), :],
                    wst.at[slot], wsem.at[slot])
            else:
                c = pltpu.make_async_copy(
                    w2_ref.at[le, pl.ds(r, WR), :],
                    wst.at[slot, :, pl.ds(0, d)], wsem.at[slot])
            c.start()
            return c

        NSLOT = 4
        copies = [None] * len(chunks)
        for i in range(NSLOT):
            copies[i] = start_copy(i, i)

        def convert(i):
            kind, le, r = chunks[i]
            slot = i % NSLOT
            copies[i].wait()
            if kind == "w1":
                w1b[le, pl.ds(r, WR), :] = wst[slot].astype(jnp.bfloat16)
            else:
                w2b[le, pl.ds(r, WR), :] = (
                    wst[slot, :, pl.ds(0, d)].astype(jnp.bfloat16))
            if i + NSLOT < len(chunks):
                copies[i + NSLOT] = start_copy(i + NSLOT, slot)

        barrier = pltpu.get_barrier_semaphore()
        for off in range(1, N_DEV):
            pl.semaphore_signal(
                barrier, inc=1,
                device_id=((my + off) % N_DEV,),
                device_id_type=pl.DeviceIdType.MESH,
            )
        pl.semaphore_wait(barrier, N_DEV - 1)

        def dispatch(dst):
            rows = lax.broadcasted_iota(jnp.int32, (CAP, t), 0) + dst * CAP
            gm = (rows == kr_ref[:]).astype(jnp.bfloat16)
            return jnp.dot(
                gm, x_ref[:], preferred_element_type=jnp.float32
            ).astype(jnp.bfloat16)

        sends = []
        for off in range(1, N_DEV):
            dst = (my + off) % N_DEV
            xsend[pl.ds(dst * CAP, CAP), :] = dispatch(dst)
            cx = pltpu.make_async_remote_copy(
                src_ref=xsend.at[pl.ds(dst * CAP, CAP)],
                dst_ref=xr.at[pl.ds(my * CAP, CAP)],
                send_sem=sx.at[off], recv_sem=rx.at[my],
                device_id=(dst,), device_id_type=pl.DeviceIdType.MESH,
            )
            cx.start()
            sends.append(cx)
        xr[pl.ds(my * CAP, CAP), :] = dispatch(my)

        def recv_wait_rows(buf, sems, s):
            rcv = pltpu.make_async_remote_copy(
                src_ref=buf.at[pl.ds(s * CAP, CAP)],
                dst_ref=buf.at[pl.ds(s * CAP, CAP)],
                send_sem=sems.at[s], recv_sem=sems.at[s],
                device_id=(my,), device_id_type=pl.DeviceIdType.MESH,
            )
            rcv.wait_recv()

        def expert_contrib(src, le):
            xs = xr[pl.ds(src * CAP + le * CAP_E, CAP_E), :]
            h = jnp.dot(xs, w1b[le], preferred_element_type=jnp.float32)
            h = jnp.maximum(h, 0.0).astype(jnp.bfloat16)
            return jnp.dot(h, w2b[le], preferred_element_type=jnp.float32)

        def gt_slice(s):
            cols = lax.broadcasted_iota(jnp.int32, (t, CAP), 1) + s * CAP
            return (cols == kc_ref[:]).astype(jnp.bfloat16)

        half = len(chunks) // N_LOC_E
        for i in range(half):
            convert(i)
        rrec[pl.ds(my * CAP, CAP_E), :] = (
            expert_contrib(my, 0).astype(jnp.bfloat16))
        for i in range(half, len(chunks)):
            convert(i)
        rrec[pl.ds(my * CAP + CAP_E, CAP_E), :] = (
            expert_contrib(my, 1).astype(jnp.bfloat16))
        out_ref[:] = jnp.dot(
            gt_slice(my), rrec[pl.ds(my * CAP, CAP), :],
            preferred_element_type=jnp.float32,
        )

        for off in range(N_DEV - 1, 0, -1):
            src = (my + off) % N_DEV
            recv_wait_rows(xr, rx, src)
            pbuf[off - 1, pl.ds(0, CAP_E), :] = (
                expert_contrib(src, 0).astype(jnp.bfloat16))
            pbuf[off - 1, pl.ds(CAP_E, CAP_E), :] = (
                expert_contrib(src, 1).astype(jnp.bfloat16))
            cp = pltpu.make_async_remote_copy(
                src_ref=pbuf.at[off - 1],
                dst_ref=rrec.at[pl.ds(my * CAP, CAP)],
                send_sem=sp.at[off], recv_sem=rp.at[my],
                device_id=(src,), device_id_type=pl.DeviceIdType.MESH,
            )
            cp.start()
            sends.append(cp)

        for off in range(1, N_DEV):
            s = (my + off) % N_DEV
            recv_wait_rows(rrec, rp, s)
            out_ref[:] = out_ref[:] + jnp.dot(
                gt_slice(s), rrec[pl.ds(s * CAP, CAP), :],
                preferred_element_type=jnp.float32,
            )

        for c in sends:
            c.wait_send()

    return pl.pallas_call(
        body,
        out_shape=jax.ShapeDtypeStruct((t, d), jnp.float32),
        in_specs=(
            [pl.BlockSpec(memory_space=pltpu.VMEM)] * 3
            + [pl.BlockSpec(memory_space=pl.ANY)] * 2
        ),
        out_specs=pl.BlockSpec(memory_space=pltpu.VMEM),
        scratch_shapes=[
            pltpu.VMEM((NC, d), jnp.bfloat16),
            pltpu.VMEM((NC, d), jnp.bfloat16),
            pltpu.VMEM((N_DEV - 1, CAP, d), jnp.bfloat16),
            pltpu.VMEM((NC, d), jnp.bfloat16),
            pltpu.VMEM((N_LOC_E, d, f), jnp.bfloat16),
            pltpu.VMEM((N_LOC_E, f, d), jnp.bfloat16),
            pltpu.VMEM((4, WR, f), jnp.float32),
            pltpu.SemaphoreType.DMA((N_DEV,)),
            pltpu.SemaphoreType.DMA((N_DEV,)),
            pltpu.SemaphoreType.DMA((N_DEV,)),
            pltpu.SemaphoreType.DMA((N_DEV,)),
            pltpu.SemaphoreType.DMA((4,)),
        ],
        compiler_params=pltpu.CompilerParams(
            collective_id=0,
            vmem_limit_bytes=56 * 1024 * 1024,
        ),
    )(xb, krow, kcol, W1, W2)


# device time: 60614 ns/iter; 2.4781x vs baseline; 1.0415x over previous
import jax
import jax.numpy as jnp
from jax import lax
from jax.experimental import pallas as pl
from jax.experimental.pallas import tpu as pltpu

N_DEV = 4
N_LOC_E = 2
N_EXP = 8
CAP_E = 160
CAP = N_LOC_E * CAP_E
NC = N_DEV * CAP


def kernel(x, assign, W1, W2):
    t, d = x.shape
    _, _, f = W1.shape
    xb = x.astype(jnp.bfloat16)

    onehot = (assign[:, None] == jnp.arange(N_EXP)[None, :]).astype(jnp.int32)
    pos = jnp.sum(onehot * jnp.cumsum(onehot, axis=0), axis=1) - 1
    k = assign * CAP_E + pos
    krow = k.reshape(1, t)
    kcol = k.reshape(t, 1)

    WR = 256

    def body(x_ref, kr_ref, kc_ref, w1_ref, w2_ref, out_ref,
             xsend, xr, pbuf, rrec, w1b, w2b, wst,
             sx, rx, sp, rp, wsem):
        my = lax.axis_index("i")

        chunks = []
        for le in range(N_LOC_E):
            for r in range(0, d, WR):
                chunks.append(("w1", le, r))
            for r in range(0, f, WR):
                chunks.append(("w2", le, r))

        def start_copy(i, slot):
            kind, le, r = chunks[i]
            if kind == "w1":
                c = pltpu.make_async_copy(
                    w1_ref.at[le, pl.ds(r, WR), :],
                    wst.at[slot], wsem.at[slot])
            else:
                c = pltpu.make_async_copy(
                    w2_ref.at[le, pl.ds(r, WR), :],
                    wst.at[slot, :, pl.ds(0, d)], wsem.at[slot])
            c.start()
            return c

        NSLOT = 4
        copies = [None] * len(chunks)
        for i in range(NSLOT):
            copies[i] = start_copy(i, i)

        def convert(i):
            kind, le, r = chunks[i]
            slot = i % NSLOT
            copies[i].wait()
            if kind == "w1":
                w1b[le, pl.ds(r, WR), :] = wst[slot].astype(jnp.bfloat16)
            else:
                w2b[le, pl.ds(r, WR), :] = (
                    wst[slot, :, pl.ds(0, d)].astype(jnp.bfloat16))
            if i + NSLOT < len(chunks):
                copies[i + NSLOT] = start_copy(i + NSLOT, slot)

        barrier = pltpu.get_barrier_semaphore()
        for off in range(1, N_DEV):
            pl.semaphore_signal(
                barrier, inc=1,
                device_id=((my + off) % N_DEV,),
                device_id_type=pl.DeviceIdType.MESH,
            )
        pl.semaphore_wait(barrier, N_DEV - 1)

        def dispatch(dst):
            rows = lax.broadcasted_iota(jnp.int32, (CAP, t), 0) + dst * CAP
            gm = (rows == kr_ref[:]).astype(jnp.bfloat16)
            return jnp.dot(
                gm, x_ref[:], preferred_element_type=jnp.float32
            ).astype(jnp.bfloat16)

        sends = []
        for off in range(1, N_DEV):
            dst = (my + off) % N_DEV
            xsend[pl.ds(dst * CAP, CAP), :] = dispatch(dst)
            cx = pltpu.make_async_remote_copy(
                src_ref=xsend.at[pl.ds(dst * CAP, CAP)],
                dst_ref=xr.at[pl.ds(my * CAP, CAP)],
                send_sem=sx.at[off], recv_sem=rx.at[my],
                device_id=(dst,), device_id_type=pl.DeviceIdType.MESH,
            )
            cx.start()
            sends.append(cx)
        xr[pl.ds(my * CAP, CAP), :] = dispatch(my)

        def recv_wait_rows(buf, sems, s):
            rcv = pltpu.make_async_remote_copy(
                src_ref=buf.at[pl.ds(s * CAP, CAP)],
                dst_ref=buf.at[pl.ds(s * CAP, CAP)],
                send_sem=sems.at[s], recv_sem=sems.at[s],
                device_id=(my,), device_id_type=pl.DeviceIdType.MESH,
            )
            rcv.wait_recv()

        def expert_contrib(src, le):
            xs = xr[pl.ds(src * CAP + le * CAP_E, CAP_E), :]
            h = jnp.dot(xs, w1b[le], preferred_element_type=jnp.float32)
            h = jnp.maximum(h, 0.0).astype(jnp.bfloat16)
            return jnp.dot(h, w2b[le], preferred_element_type=jnp.float32)

        def gt_slice(s):
            cols = lax.broadcasted_iota(jnp.int32, (t, CAP), 1) + s * CAP
            return (cols == kc_ref[:]).astype(jnp.bfloat16)

        half = len(chunks) // N_LOC_E
        for i in range(half):
            convert(i)
        rrec[pl.ds(my * CAP, CAP_E), :] = (
            expert_contrib(my, 0).astype(jnp.bfloat16))
        for i in range(half, len(chunks)):
            convert(i)
        rrec[pl.ds(my * CAP + CAP_E, CAP_E), :] = (
            expert_contrib(my, 1).astype(jnp.bfloat16))
        out_ref[:] = jnp.dot(
            gt_slice(my), rrec[pl.ds(my * CAP, CAP), :],
            preferred_element_type=jnp.float32,
        )

        for off in range(N_DEV - 1, 0, -1):
            src = (my + off) % N_DEV
            recv_wait_rows(xr, rx, src)
            pbuf[off - 1, pl.ds(0, CAP_E), :] = (
                expert_contrib(src, 0).astype(jnp.bfloat16))
            pbuf[off - 1, pl.ds(CAP_E, CAP_E), :] = (
                expert_contrib(src, 1).astype(jnp.bfloat16))
            cp = pltpu.make_async_remote_copy(
                src_ref=pbuf.at[off - 1],
                dst_ref=rrec.at[pl.ds(my * CAP, CAP)],
                send_sem=sp.at[off], recv_sem=rp.at[my],
                device_id=(src,), device_id_type=pl.DeviceIdType.MESH,
            )
            cp.start()
            sends.append(cp)

        for off in range(1, N_DEV):
            s = (my + off) % N_DEV
            recv_wait_rows(rrec, rp, s)
            out_ref[:] = out_ref[:] + jnp.dot(
                gt_slice(s), rrec[pl.ds(s * CAP, CAP), :],
                preferred_element_type=jnp.float32,
            )

        for c in sends:
            c.wait_send()

    return pl.pallas_call(
        body,
        out_shape=jax.ShapeDtypeStruct((t, d), jnp.float32),
        in_specs=(
            [pl.BlockSpec(memory_space=pltpu.VMEM)] * 3
            + [pl.BlockSpec(memory_space=pl.ANY)] * 2
        ),
        out_specs=pl.BlockSpec(memory_space=pltpu.VMEM),
        scratch_shapes=[
            pltpu.VMEM((NC, d), jnp.bfloat16),
            pltpu.VMEM((NC, d), jnp.bfloat16),
            pltpu.VMEM((N_DEV - 1, CAP, d), jnp.bfloat16),
            pltpu.VMEM((NC, d), jnp.bfloat16),
            pltpu.VMEM((N_LOC_E, d, f), jnp.bfloat16),
            pltpu.VMEM((N_LOC_E, f, d), jnp.bfloat16),
            pltpu.VMEM((4, WR, f), jnp.float32),
            pltpu.SemaphoreType.DMA((N_DEV,)),
            pltpu.SemaphoreType.DMA((N_DEV,)),
            pltpu.SemaphoreType.DMA((N_DEV,)),
            pltpu.SemaphoreType.DMA((N_DEV,)),
            pltpu.SemaphoreType.DMA((4,)),
        ],
        compiler_params=pltpu.CompilerParams(
            collective_id=0,
            vmem_limit_bytes=56 * 1024 * 1024,
        ),
    )(xb, krow, kcol, W1, W2)
